# baseline (speedup 1.0000x reference)
import jax
import jax.numpy as jnp
from jax import lax
from jax.experimental import pallas as pl
from jax.experimental.pallas import tpu as pltpu
from jax.experimental.pallas import tpu_sc as plsc

N_FIELDS = 26
FIELD_DIM = 100000
N_FACTORS = 32
BATCH = 16384
N_FEAT = N_FIELDS * FIELD_DIM

NC, NS = 2, 16
NW = NC * NS
ROWS_PER_W = BATCH // NW
CHUNK = 16
N_CHUNKS = ROWS_PER_W // CHUNK
IDXC = CHUNK * N_FIELDS
X_PER_W = ROWS_PER_W * N_FIELDS


def _fm_body(x_hbm, emb_hbm, fc_hbm, w_hbm, b_hbm, out_hbm,
             xs0, xs1, idx0, idx1, cb0, cb1, gix0, gix1,
             rows0, rows1, fcv0, fcv1,
             outv, wv_v, bv_v, sem0, sem1):
    cid = lax.axis_index("c")
    sid = lax.axis_index("s")
    wid = sid * NC + cid
    xbase = wid * X_PER_W

    pltpu.sync_copy(w_hbm, wv_v)
    pltpu.sync_copy(b_hbm, bv_v)
    Wv = wv_v[...]
    Bv = bv_v[...]

    iota16 = jnp.arange(16, dtype=jnp.int32)
    riota26 = iota16 * N_FIELDS

    xs_bufs = (xs0, xs1)
    idx_bufs = (idx0, idx1)
    cb_bufs = (cb0, cb1)
    gix_bufs = (gix0, gix1)
    rows_bufs = (rows0, rows1)
    fc_bufs = (fcv0, fcv1)
    sems = (sem0, sem1)

    def prep_fire(ci, p):
        xsP, idxP, cbP, gixP = xs_bufs[p], idx_bufs[p], cb_bufs[p], gix_bufs[p]
        rowsP, fcP, semP = rows_bufs[p], fc_bufs[p], sems[p]

        pltpu.sync_copy(x_hbm.at[pl.ds(xbase + ci * IDXC, IDXC)], xsP)

        def kbody(k, _):
            off = k * 16
            pos = off + iota16
            xv = xsP[pl.ds(off, 16)]
            fld = lax.rem(pos, N_FIELDS)
            gidx = xv + fld * FIELD_DIM
            gixP[pl.ds(off, 16)] = gidx
            idxP[pl.ds(off, 16)] = lax.shift_right_logical(gidx, 2)
            cbP[pl.ds(off, 16)] = lax.shift_left(
                lax.bitwise_and(gidx, 3), 5)
            return 0

        lax.fori_loop(0, IDXC // 16, kbody, 0)
        for j in range(4):
            lo = j * 128
            n = min(128, IDXC - lo)
            pltpu.async_copy(emb_hbm.at[idxP.at[pl.ds(lo, n)]],
                             rowsP.at[pl.ds(lo, n)], semP)
            pltpu.async_copy(fc_hbm.at[gixP.at[pl.ds(lo, n)]],
                             fcP.at[pl.ds(lo, n)], semP)

    def drain(p):
        pltpu.make_async_copy(emb_hbm.at[pl.ds(0, IDXC)],
                              rows_bufs[p], sems[p]).wait()
        pltpu.make_async_copy(fc_hbm.at[pl.ds(0, IDXC)],
                              fc_bufs[p], sems[p]).wait()

    def compute(ci, p):
        rowsP, fcP, cbP = rows_bufs[p], fc_bufs[p], cb_bufs[p]
        zf = jnp.zeros((16,), jnp.float32)

        posv = [riota26 + f for f in range(N_FIELDS)]
        cbv = [plsc.load_gather(cbP, [posv[f]]) for f in range(N_FIELDS)]

        def cbody(c, carry):
            q0, q1, q2, q3, nrm = carry
            s0, s1 = zf, zf
            for f in range(N_FIELDS):
                e = plsc.load_gather(rowsP, [posv[f], cbv[f] + c])
                t = e * e
                if f % 2 == 0:
                    s0 = s0 + e
                else:
                    s1 = s1 + e
                if f % 4 == 0:
                    q0 = q0 + t
                elif f % 4 == 1:
                    q1 = q1 + t
                elif f % 4 == 2:
                    q2 = q2 + t
                else:
                    q3 = q3 + t
            s = s0 + s1
            nrm = nrm + s * s
            return (q0, q1, q2, q3, nrm)

        q0, q1, q2, q3, nrm = lax.fori_loop(
            0, N_FACTORS, cbody, (zf, zf, zf, zf, zf))

        f0, f1 = zf, zf
        for f in range(N_FIELDS):
            e = plsc.load_gather(fcP, [posv[f]])
            if f % 2 == 0:
                f0 = f0 + e
            else:
                f1 = f1 + e

        res = (f0 + f1) * Wv + Bv + 0.5 * (nrm - (q0 + q1) - (q2 + q3))
        outv[pl.ds(ci * CHUNK, 16)] = res

    prep_fire(0, 0)

    def pair(i, _):
        i0 = 2 * i
        i1 = i0 + 1
        prep_fire(i1, 1)
        drain(0)
        compute(i0, 0)

        @pl.when(i1 + 1 < N_CHUNKS)
        def _():
            prep_fire(i1 + 1, 0)

        drain(1)
        compute(i1, 1)
        return 0

    lax.fori_loop(0, N_CHUNKS // 2, pair, 0)

    pltpu.sync_copy(outv, out_hbm.at[pl.ds(wid * ROWS_PER_W, ROWS_PER_W)])


_fm_sc = pl.kernel(
    _fm_body,
    out_type=jax.ShapeDtypeStruct((BATCH,), jnp.float32),
    mesh=plsc.VectorSubcoreMesh(core_axis_name="c", subcore_axis_name="s"),
    compiler_params=pltpu.CompilerParams(needs_layout_passes=False,
                                         use_tc_tiling_on_sc=False),
    scratch_types=[
        pltpu.VMEM((IDXC,), jnp.int32),
        pltpu.VMEM((IDXC,), jnp.int32),
        pltpu.VMEM((IDXC,), jnp.int32),
        pltpu.VMEM((IDXC,), jnp.int32),
        pltpu.VMEM((IDXC,), jnp.int32),
        pltpu.VMEM((IDXC,), jnp.int32),
        pltpu.VMEM((IDXC,), jnp.int32),
        pltpu.VMEM((IDXC,), jnp.int32),
        pltpu.VMEM((IDXC, 128), jnp.float32),
        pltpu.VMEM((IDXC, 128), jnp.float32),
        pltpu.VMEM((IDXC,), jnp.float32),
        pltpu.VMEM((IDXC,), jnp.float32),
        pltpu.VMEM((ROWS_PER_W,), jnp.float32),
        pltpu.VMEM((16,), jnp.float32),
        pltpu.VMEM((16,), jnp.float32),
        pltpu.SemaphoreType.DMA,
        pltpu.SemaphoreType.DMA,
    ],
)


@jax.jit
def kernel(x, embedding, fc, W, b):
    x_flat = x.astype(jnp.int32).reshape(-1)
    emb4 = embedding.reshape(N_FEAT // 4, N_FACTORS * 4)
    fc_flat = fc.reshape(-1).astype(jnp.float32)
    wv = jnp.full((16,), W[0, 0], dtype=jnp.float32)
    bv = jnp.full((16,), b[0], dtype=jnp.float32)
    return _fm_sc(x_flat, emb4, fc_flat, wv, bv)


# speedup vs baseline: 1.4892x; 1.4892x over previous
import jax
import jax.numpy as jnp
from jax import lax
from jax.experimental import pallas as pl
from jax.experimental.pallas import tpu as pltpu
from jax.experimental.pallas import tpu_sc as plsc

N_FIELDS = 26
FIELD_DIM = 100000
N_FACTORS = 32
BATCH = 16384
N_FEAT = N_FIELDS * FIELD_DIM

NC, NS = 2, 16
NW = NC * NS

N_TILE_COLS = N_FEAT // 128
TAIL = N_FEAT - N_TILE_COLS * 128
BLK_HI = 635
N_HI = 24
BLK_LO = 634


def _tr_body(embt_hbm, tail_hbm, out_hbm, in0, in1, ot0, ot1,
             si0, si1, so0, so1):
    cid = lax.axis_index("c")
    sid = lax.axis_index("s")
    wid = sid * NC + cid

    n = lax.select(wid < N_HI, BLK_HI, BLK_LO)
    start = lax.select(wid < N_HI, wid * BLK_HI,
                       N_HI * BLK_HI + (wid - N_HI) * BLK_LO)

    iota16 = jnp.arange(16, dtype=jnp.int32)
    in_bufs = (in0, in1)
    out_bufs = (ot0, ot1)
    sin = (si0, si1)
    sout = (so0, so1)

    def fire_load(t, p):
        pltpu.async_copy(embt_hbm.at[:, pl.ds(t * 128, 128)],
                         in_bufs[p], sin[p])

    def wait_load(p):
        pltpu.make_async_copy(embt_hbm.at[:, pl.ds(0, 128)],
                              in_bufs[p], sin[p]).wait()

    def fire_store(t, p):
        pltpu.async_copy(out_bufs[p], out_hbm.at[pl.ds(t * 32, 32), :],
                         sout[p])

    def wait_store(p):
        pltpu.make_async_copy(out_bufs[p],
                              out_hbm.at[pl.ds(0, 32), :], sout[p]).wait()

    def transpose(p, nrows16):
        inb, otb = in_bufs[p], out_bufs[p]

        def pass_s(s, _):
            rot = lax.bitwise_and(iota16 + s, 15)
            for c0 in (0, 16):
                civ = c0 + iota16

                def pass_r(r0, _):
                    r = r0 * 16 + rot
                    v = plsc.load_gather(inb, [civ, r])
                    out_r = lax.shift_right_logical(r, 2)
                    out_c = lax.shift_left(lax.bitwise_and(r, 3), 5) + civ
                    plsc.store_scatter(otb, [out_r, out_c], v)
                    return 0

                lax.fori_loop(0, nrows16, pass_r, 0)
            return 0

        lax.fori_loop(0, 16, pass_s, 0)

    fire_load(start, 0)
    n_pairs = lax.shift_right_logical(n + 1, 1)

    def pair(i, _):
        i0 = 2 * i
        t0 = start + i0

        @pl.when(i0 + 1 < n)
        def _():
            fire_load(t0 + 1, 1)

        wait_load(0)

        @pl.when(i0 >= 2)
        def _():
            wait_store(0)

        transpose(0, 8)
        fire_store(t0, 0)

        @pl.when(i0 + 1 < n)
        def _():
            @pl.when(i0 + 2 < n)
            def _():
                fire_load(t0 + 2, 0)

            wait_load(1)

            @pl.when(i0 >= 1)
            def _():
                wait_store(1)

            transpose(1, 8)
            fire_store(t0 + 1, 1)

        return 0

    lax.fori_loop(0, n_pairs, pair, 0)
    wait_store(0)
    wait_store(1)

    @pl.when(wid == NW - 1)
    def _():
        pltpu.sync_copy(tail_hbm, in0.at[pl.ds(0, TAIL // 4)])
        pltpu.sync_copy(in0.at[pl.ds(0, TAIL // 4)],
                        out_hbm.at[pl.ds(N_TILE_COLS * 32, TAIL // 4)])


_tr_sc = pl.kernel(
    _tr_body,
    out_type=jax.ShapeDtypeStruct((N_FEAT // 4, 128), jnp.float32),
    mesh=plsc.VectorSubcoreMesh(core_axis_name="c", subcore_axis_name="s"),
    compiler_params=pltpu.CompilerParams(needs_layout_passes=False,
                                         use_tc_tiling_on_sc=True),
    scratch_types=[
        pltpu.VMEM((N_FACTORS, 128), jnp.float32),
        pltpu.VMEM((N_FACTORS, 128), jnp.float32),
        pltpu.VMEM((N_FACTORS, 128), jnp.float32),
        pltpu.VMEM((N_FACTORS, 128), jnp.float32),
        pltpu.SemaphoreType.DMA,
        pltpu.SemaphoreType.DMA,
        pltpu.SemaphoreType.DMA,
        pltpu.SemaphoreType.DMA,
    ],
)


ROWS_PER_W = BATCH // NW
CHUNK = 16
N_CHUNKS = ROWS_PER_W // CHUNK
IDXC = CHUNK * N_FIELDS
X_PER_W = ROWS_PER_W * N_FIELDS


def _fm_body(x_hbm, emb_hbm, fc_hbm, w_hbm, b_hbm, out_hbm,
             xs0, xs1, idx0, idx1, cb0, cb1, gix0, gix1,
             rows0, rows1, fcv0, fcv1,
             outv, wv_v, bv_v, sem0, sem1):
    cid = lax.axis_index("c")
    sid = lax.axis_index("s")
    wid = sid * NC + cid
    xbase = wid * X_PER_W

    pltpu.sync_copy(w_hbm, wv_v)
    pltpu.sync_copy(b_hbm, bv_v)
    Wv = wv_v[...]
    Bv = bv_v[...]

    iota16 = jnp.arange(16, dtype=jnp.int32)
    riota26 = iota16 * N_FIELDS

    xs_bufs = (xs0, xs1)
    idx_bufs = (idx0, idx1)
    cb_bufs = (cb0, cb1)
    gix_bufs = (gix0, gix1)
    rows_bufs = (rows0, rows1)
    fc_bufs = (fcv0, fcv1)
    sems = (sem0, sem1)

    def prep_fire(ci, p):
        xsP, idxP, cbP, gixP = xs_bufs[p], idx_bufs[p], cb_bufs[p], gix_bufs[p]
        rowsP, fcP, semP = rows_bufs[p], fc_bufs[p], sems[p]

        pltpu.sync_copy(x_hbm.at[pl.ds(xbase + ci * IDXC, IDXC)], xsP)

        def kbody(k, _):
            off = k * 16
            pos = off + iota16
            xv = xsP[pl.ds(off, 16)]
            fld = lax.rem(pos, N_FIELDS)
            gidx = xv + fld * FIELD_DIM
            gixP[pl.ds(off, 16)] = gidx
            idxP[pl.ds(off, 16)] = lax.shift_right_logical(gidx, 2)
            cbP[pl.ds(off, 16)] = lax.shift_left(
                lax.bitwise_and(gidx, 3), 5)
            return 0

        lax.fori_loop(0, IDXC // 16, kbody, 0)
        for j in range(4):
            lo = j * 128
            m = min(128, IDXC - lo)
            pltpu.async_copy(emb_hbm.at[idxP.at[pl.ds(lo, m)]],
                             rowsP.at[pl.ds(lo, m)], semP)
            pltpu.async_copy(fc_hbm.at[gixP.at[pl.ds(lo, m)]],
                             fcP.at[pl.ds(lo, m)], semP)

    def drain(p):
        pltpu.make_async_copy(emb_hbm.at[pl.ds(0, IDXC)],
                              rows_bufs[p], sems[p]).wait()
        pltpu.make_async_copy(fc_hbm.at[pl.ds(0, IDXC)],
                              fc_bufs[p], sems[p]).wait()

    def compute(ci, p):
        rowsP, fcP, cbP = rows_bufs[p], fc_bufs[p], cb_bufs[p]
        zf = jnp.zeros((16,), jnp.float32)

        posv = [riota26 + f for f in range(N_FIELDS)]
        cbv = [plsc.load_gather(cbP, [posv[f]]) for f in range(N_FIELDS)]

        def cbody(c, carry):
            q0, q1, q2, q3, nrm = carry
            s0, s1 = zf, zf
            for f in range(N_FIELDS):
                e = plsc.load_gather(rowsP, [posv[f], cbv[f] + c])
                t = e * e
                if f % 2 == 0:
                    s0 = s0 + e
                else:
                    s1 = s1 + e
                if f % 4 == 0:
                    q0 = q0 + t
                elif f % 4 == 1:
                    q1 = q1 + t
                elif f % 4 == 2:
                    q2 = q2 + t
                else:
                    q3 = q3 + t
            s = s0 + s1
            nrm = nrm + s * s
            return (q0, q1, q2, q3, nrm)

        q0, q1, q2, q3, nrm = lax.fori_loop(
            0, N_FACTORS, cbody, (zf, zf, zf, zf, zf))

        f0, f1 = zf, zf
        for f in range(N_FIELDS):
            e = plsc.load_gather(fcP, [posv[f]])
            if f % 2 == 0:
                f0 = f0 + e
            else:
                f1 = f1 + e

        res = (f0 + f1) * Wv + Bv + 0.5 * (nrm - (q0 + q1) - (q2 + q3))
        outv[pl.ds(ci * CHUNK, 16)] = res

    prep_fire(0, 0)

    def pair(i, _):
        i0 = 2 * i
        i1 = i0 + 1
        prep_fire(i1, 1)
        drain(0)
        compute(i0, 0)

        @pl.when(i1 + 1 < N_CHUNKS)
        def _():
            prep_fire(i1 + 1, 0)

        drain(1)
        compute(i1, 1)
        return 0

    lax.fori_loop(0, N_CHUNKS // 2, pair, 0)

    pltpu.sync_copy(outv, out_hbm.at[pl.ds(wid * ROWS_PER_W, ROWS_PER_W)])


_fm_sc = pl.kernel(
    _fm_body,
    out_type=jax.ShapeDtypeStruct((BATCH,), jnp.float32),
    mesh=plsc.VectorSubcoreMesh(core_axis_name="c", subcore_axis_name="s"),
    compiler_params=pltpu.CompilerParams(needs_layout_passes=False,
                                         use_tc_tiling_on_sc=False),
    scratch_types=[
        pltpu.VMEM((IDXC,), jnp.int32),
        pltpu.VMEM((IDXC,), jnp.int32),
        pltpu.VMEM((IDXC,), jnp.int32),
        pltpu.VMEM((IDXC,), jnp.int32),
        pltpu.VMEM((IDXC,), jnp.int32),
        pltpu.VMEM((IDXC,), jnp.int32),
        pltpu.VMEM((IDXC,), jnp.int32),
        pltpu.VMEM((IDXC,), jnp.int32),
        pltpu.VMEM((IDXC, 128), jnp.float32),
        pltpu.VMEM((IDXC, 128), jnp.float32),
        pltpu.VMEM((IDXC,), jnp.float32),
        pltpu.VMEM((IDXC,), jnp.float32),
        pltpu.VMEM((ROWS_PER_W,), jnp.float32),
        pltpu.VMEM((16,), jnp.float32),
        pltpu.VMEM((16,), jnp.float32),
        pltpu.SemaphoreType.DMA,
        pltpu.SemaphoreType.DMA,
    ],
)


_DEBUG_BISECT = False


@jax.jit
def kernel(x, embedding, fc, W, b):
    x_flat = x.astype(jnp.int32).reshape(-1)
    tail4 = embedding[N_TILE_COLS * 128:].reshape(TAIL // 4, 128)
    emb4 = _tr_sc(embedding.T, tail4)
    if _DEBUG_BISECT:
        emb_rec = emb4.reshape(N_FEAT, N_FACTORS)
        offsets = (jnp.arange(N_FIELDS, dtype=jnp.int32) * FIELD_DIM)[None, :]
        idx = x.astype(jnp.int32) + offsets
        embv = jnp.take(emb_rec, idx, axis=0)
        sqs = jnp.sum(embv, axis=1) ** 2
        sos = jnp.sum(embv ** 2, axis=1)
        lin = jnp.take(fc, idx, axis=0).sum(axis=1) @ W.T + b
        return (lin + 0.5 * jnp.sum(sqs - sos, axis=1, keepdims=True)).reshape(-1)
    fc_flat = fc.reshape(-1).astype(jnp.float32)
    wv = jnp.full((16,), W[0, 0], dtype=jnp.float32)
    bv = jnp.full((16,), b[0], dtype=jnp.float32)
    return _fm_sc(x_flat, emb4, fc_flat, wv, bv)


# speedup vs baseline: 2.1990x; 1.4766x over previous
import jax
import jax.numpy as jnp
from jax import lax
from jax.experimental import pallas as pl
from jax.experimental.pallas import tpu as pltpu
from jax.experimental.pallas import tpu_sc as plsc

N_FIELDS = 26
FIELD_DIM = 100000
N_FACTORS = 32
BATCH = 16384
N_FEAT = N_FIELDS * FIELD_DIM

NC, NS = 2, 16
NW = NC * NS

N_TILE_COLS = N_FEAT // 128
TAIL = N_FEAT - N_TILE_COLS * 128
BLK_HI = 635
N_HI = 24
BLK_LO = 634


def _tr_body(embt_hbm, tail_hbm, out_hbm, in0, in1, ot0, ot1,
             si0, si1, so0, so1):
    cid = lax.axis_index("c")
    sid = lax.axis_index("s")
    wid = sid * NC + cid

    n = lax.select(wid < N_HI, BLK_HI, BLK_LO)
    start = lax.select(wid < N_HI, wid * BLK_HI,
                       N_HI * BLK_HI + (wid - N_HI) * BLK_LO)

    iota16 = jnp.arange(16, dtype=jnp.int32)
    in_bufs = (in0, in1)
    out_bufs = (ot0, ot1)
    sin = (si0, si1)
    sout = (so0, so1)

    def fire_load(t, p):
        pltpu.async_copy(embt_hbm.at[:, pl.ds(t * 128, 128)],
                         in_bufs[p], sin[p])

    def wait_load(p):
        pltpu.make_async_copy(embt_hbm.at[:, pl.ds(0, 128)],
                              in_bufs[p], sin[p]).wait()

    def fire_store(t, p):
        pltpu.async_copy(out_bufs[p], out_hbm.at[pl.ds(t * 32, 32), :],
                         sout[p])

    def wait_store(p):
        pltpu.make_async_copy(out_bufs[p],
                              out_hbm.at[pl.ds(0, 32), :], sout[p]).wait()

    def transpose(p):
        inb, otb = in_bufs[p], out_bufs[p]

        def pass_s(s, _):
            rot = lax.bitwise_and(iota16 + s, 15)
            rsh = lax.shift_right_logical(rot, 2)
            rc = lax.shift_left(lax.bitwise_and(rot, 3), 5)
            for c0 in (0, 16):
                civ = c0 + iota16
                ocv = rc + civ
                for r0 in range(8):
                    r = r0 * 16 + rot
                    v = plsc.load_gather(inb, [civ, r])
                    plsc.store_scatter(otb, [rsh + r0 * 4, ocv], v)
            return 0

        lax.fori_loop(0, 16, pass_s, 0)

    fire_load(start, 0)
    n_pairs = lax.shift_right_logical(n + 1, 1)

    def pair(i, _):
        i0 = 2 * i
        t0 = start + i0

        @pl.when(i0 + 1 < n)
        def _():
            fire_load(t0 + 1, 1)

        wait_load(0)

        @pl.when(i0 >= 2)
        def _():
            wait_store(0)

        transpose(0)
        fire_store(t0, 0)

        @pl.when(i0 + 1 < n)
        def _():
            @pl.when(i0 + 2 < n)
            def _():
                fire_load(t0 + 2, 0)

            wait_load(1)

            @pl.when(i0 >= 1)
            def _():
                wait_store(1)

            transpose(1)
            fire_store(t0 + 1, 1)

        return 0

    lax.fori_loop(0, n_pairs, pair, 0)
    wait_store(0)
    wait_store(1)

    @pl.when(wid == NW - 1)
    def _():
        pltpu.sync_copy(tail_hbm, in0.at[pl.ds(0, TAIL // 4)])
        pltpu.sync_copy(in0.at[pl.ds(0, TAIL // 4)],
                        out_hbm.at[pl.ds(N_TILE_COLS * 32, TAIL // 4)])


_tr_sc = pl.kernel(
    _tr_body,
    out_type=jax.ShapeDtypeStruct((N_FEAT // 4, 128), jnp.float32),
    mesh=plsc.VectorSubcoreMesh(core_axis_name="c", subcore_axis_name="s"),
    compiler_params=pltpu.CompilerParams(needs_layout_passes=False,
                                         use_tc_tiling_on_sc=True),
    scratch_types=[
        pltpu.VMEM((N_FACTORS, 128), jnp.float32),
        pltpu.VMEM((N_FACTORS, 128), jnp.float32),
        pltpu.VMEM((N_FACTORS, 128), jnp.float32),
        pltpu.VMEM((N_FACTORS, 128), jnp.float32),
        pltpu.SemaphoreType.DMA,
        pltpu.SemaphoreType.DMA,
        pltpu.SemaphoreType.DMA,
        pltpu.SemaphoreType.DMA,
    ],
)


ROWS_PER_W = BATCH // NW
CHUNK = 64
N_CHUNKS = ROWS_PER_W // CHUNK
IDXC = CHUNK * N_FIELDS
GATHERS = IDXC // 128
X_PER_W = ROWS_PER_W * N_FIELDS


def _fm_body(x_hbm, emb_hbm, fc_hbm, w_hbm, b_hbm, out_hbm,
             xall, idx0, idx1, rows0, rows1, fcv0, fcv1,
             sqv, sqq, outv, wv_v, bv_v, sem0, sem1):
    cid = lax.axis_index("c")
    sid = lax.axis_index("s")
    wid = sid * NC + cid
    xbase = wid * X_PER_W

    pltpu.sync_copy(x_hbm.at[pl.ds(xbase, X_PER_W)], xall)
    pltpu.sync_copy(w_hbm, wv_v)
    pltpu.sync_copy(b_hbm, bv_v)
    Wv = wv_v[...]
    Bv = bv_v[...]

    iota16 = jnp.arange(16, dtype=jnp.int32)
    riota26 = iota16 * N_FIELDS
    riota17 = iota16 * 17

    idx_bufs = (idx0, idx1)
    rows_bufs = (rows0, rows1)
    fc_bufs = (fcv0, fcv1)
    sems = (sem0, sem1)

    def prep_fire(ci, p):
        idxP, rowsP, fcP, semP = idx_bufs[p], rows_bufs[p], fc_bufs[p], sems[p]

        def kbody(k, _):
            off = k * 16
            pos = off + iota16
            xv = xall[pl.ds(ci * IDXC + off, 16)]
            fld = lax.rem(pos, N_FIELDS)
            idxP[pl.ds(off, 16)] = xv + fld * FIELD_DIM
            return 0

        lax.fori_loop(0, IDXC // 16, kbody, 0)
        for j in range(GATHERS):
            isl = idxP.at[pl.ds(j * 128, 128)]
            pltpu.async_copy(emb_hbm.at[isl], rowsP.at[pl.ds(j * 128, 128)], semP)
            pltpu.async_copy(fc_hbm.at[isl], fcP.at[pl.ds(j * 128, 128)], semP)

    def drain(p):
        pltpu.make_async_copy(emb_hbm.at[pl.ds(0, IDXC)],
                              rows_bufs[p], sems[p]).wait()
        pltpu.make_async_copy(fc_hbm.at[pl.ds(0, IDXC)],
                              fc_bufs[p], sems[p]).wait()

    def compute(ci, p):
        rowsP, fcP = rows_bufs[p], fc_bufs[p]
        zf = jnp.zeros((16,), jnp.float32)

        def gbody(g, _):
            gb = g * 16

            def sbody(i, _):
                sb = (gb + i) * N_FIELDS
                a0, a1, a2, a3 = zf, zf, zf, zf
                q0, q1, q2, q3 = zf, zf, zf, zf
                for f in range(N_FIELDS):
                    e0 = rowsP[sb + f, pl.ds(0, 16)]
                    e1 = rowsP[sb + f, pl.ds(16, 16)]
                    if f % 2 == 0:
                        a0 = a0 + e0
                        a1 = a1 + e1
                        q0 = q0 + e0 * e0
                        q1 = q1 + e1 * e1
                    else:
                        a2 = a2 + e0
                        a3 = a3 + e1
                        q2 = q2 + e0 * e0
                        q3 = q3 + e1 * e1
                s0 = a0 + a2
                s1 = a1 + a3
                v = s0 * s0 + s1 * s1
                q = (q0 + q1) + (q2 + q3)
                sqv[pl.ds(i * 17, 16)] = v
                sqq[pl.ds(i * 17, 16)] = q
                return 0

            lax.fori_loop(0, 16, sbody, 0)

            nrm0, nrm1, qt0, qt1 = zf, zf, zf, zf
            for k in range(16):
                ik = riota17 + k
                if k % 2 == 0:
                    nrm0 = nrm0 + plsc.load_gather(sqv, [ik])
                    qt0 = qt0 + plsc.load_gather(sqq, [ik])
                else:
                    nrm1 = nrm1 + plsc.load_gather(sqv, [ik])
                    qt1 = qt1 + plsc.load_gather(sqq, [ik])

            rowb26 = riota26 + gb * N_FIELDS
            f0, f1 = zf, zf
            for f in range(N_FIELDS):
                e = plsc.load_gather(fcP, [rowb26 + f])
                if f % 2 == 0:
                    f0 = f0 + e
                else:
                    f1 = f1 + e

            res = ((f0 + f1) * Wv + Bv
                   + 0.5 * ((nrm0 + nrm1) - (qt0 + qt1)))
            outv[pl.ds(ci * CHUNK + g * 16, 16)] = res
            return 0

        lax.fori_loop(0, CHUNK // 16, gbody, 0)

    prep_fire(0, 0)
    for ci in range(N_CHUNKS):
        p = ci & 1
        if ci + 1 < N_CHUNKS:
            prep_fire(ci + 1, 1 - p)
        drain(p)
        compute(ci, p)

    pltpu.sync_copy(outv, out_hbm.at[pl.ds(wid * ROWS_PER_W, ROWS_PER_W)])


_fm_sc = pl.kernel(
    _fm_body,
    out_type=jax.ShapeDtypeStruct((BATCH,), jnp.float32),
    mesh=plsc.VectorSubcoreMesh(core_axis_name="c", subcore_axis_name="s"),
    compiler_params=pltpu.CompilerParams(needs_layout_passes=False,
                                         use_tc_tiling_on_sc=False),
    scratch_types=[
        pltpu.VMEM((X_PER_W,), jnp.int32),
        pltpu.VMEM((IDXC,), jnp.int32),
        pltpu.VMEM((IDXC,), jnp.int32),
        pltpu.VMEM((IDXC, N_FACTORS), jnp.float32),
        pltpu.VMEM((IDXC, N_FACTORS), jnp.float32),
        pltpu.VMEM((IDXC,), jnp.float32),
        pltpu.VMEM((IDXC,), jnp.float32),
        pltpu.VMEM((16 * 17,), jnp.float32),
        pltpu.VMEM((16 * 17,), jnp.float32),
        pltpu.VMEM((ROWS_PER_W,), jnp.float32),
        pltpu.VMEM((16,), jnp.float32),
        pltpu.VMEM((16,), jnp.float32),
        pltpu.SemaphoreType.DMA,
        pltpu.SemaphoreType.DMA,
    ],
)


@jax.jit
def kernel(x, embedding, fc, W, b):
    x_flat = x.astype(jnp.int32).reshape(-1)
    tail4 = embedding[N_TILE_COLS * 128:].reshape(TAIL // 4, 128)
    emb4 = _tr_sc(embedding.T, tail4)
    emb_rows = emb4.reshape(N_FEAT, N_FACTORS)
    fc_flat = fc.reshape(-1).astype(jnp.float32)
    wv = jnp.full((16,), W[0, 0], dtype=jnp.float32)
    bv = jnp.full((16,), b[0], dtype=jnp.float32)
    return _fm_sc(x_flat, emb_rows, fc_flat, wv, bv)


# speedup vs baseline: 4.2140x; 1.9163x over previous
import jax
import jax.numpy as jnp
from jax import lax
from jax.experimental import pallas as pl
from jax.experimental.pallas import tpu as pltpu
from jax.experimental.pallas import tpu_sc as plsc

N_FIELDS = 26
FIELD_DIM = 100000
N_FACTORS = 32
BATCH = 16384
N_FEAT = N_FIELDS * FIELD_DIM

NC, NS = 2, 16
NW = NC * NS

N_TILE_COLS = N_FEAT // 128
TAIL = N_FEAT - N_TILE_COLS * 128
N_SLABS = N_TILE_COLS // 2
BLK_HI = 318
N_HI = 12
BLK_LO = 317


def _tr_body(embt_hbm, tail_hbm, out_hbm, in0, in1, ot0, ot1,
             si0, si1, so0, so1):
    cid = lax.axis_index("c")
    sid = lax.axis_index("s")
    wid = sid * NC + cid

    n = lax.select(wid < N_HI, BLK_HI, BLK_LO)
    start = lax.select(wid < N_HI, wid * BLK_HI,
                       N_HI * BLK_HI + (wid - N_HI) * BLK_LO)

    iota16 = jnp.arange(16, dtype=jnp.int32)
    in_bufs = (in0, in1)
    out_bufs = (ot0, ot1)
    sin = (si0, si1)
    sout = (so0, so1)

    def fire_load(t, p):
        pltpu.async_copy(embt_hbm.at[:, pl.ds(t * 256, 256)],
                         in_bufs[p], sin[p])

    def wait_load(p):
        pltpu.make_async_copy(embt_hbm.at[:, pl.ds(0, 256)],
                              in_bufs[p], sin[p]).wait()

    def fire_store(t, p):
        pltpu.async_copy(out_bufs[p], out_hbm.at[pl.ds(t * 64, 64), :],
                         sout[p])

    def wait_store(p):
        pltpu.make_async_copy(out_bufs[p],
                              out_hbm.at[pl.ds(0, 64), :], sout[p]).wait()

    def transpose(p):
        inb, otb = in_bufs[p], out_bufs[p]

        def pass_s(s, _):
            rot = lax.bitwise_and(iota16 + s, 15)
            rsh = lax.shift_right_logical(rot, 2)
            rc = lax.shift_left(lax.bitwise_and(rot, 3), 5)
            for u in (0, 1):
                for c0 in (0, 16):
                    civ = c0 + iota16
                    ocv = rc + civ
                    vs = []
                    for r0 in range(8):
                        r = u * 128 + r0 * 16 + rot
                        vs.append(plsc.load_gather(inb, [civ, r]))
                    for r0 in range(8):
                        plsc.store_scatter(
                            otb, [rsh + (u * 32 + r0 * 4), ocv], vs[r0])
            return 0

        lax.fori_loop(0, 16, pass_s, 0)

    fire_load(start, 0)
    n_pairs = lax.shift_right_logical(n + 1, 1)

    def pair(i, _):
        i0 = 2 * i
        t0 = start + i0

        @pl.when(i0 + 1 < n)
        def _():
            fire_load(t0 + 1, 1)

        wait_load(0)

        @pl.when(i0 >= 2)
        def _():
            wait_store(0)

        transpose(0)
        fire_store(t0, 0)

        @pl.when(i0 + 1 < n)
        def _():
            @pl.when(i0 + 2 < n)
            def _():
                fire_load(t0 + 2, 0)

            wait_load(1)

            @pl.when(i0 >= 1)
            def _():
                wait_store(1)

            transpose(1)
            fire_store(t0 + 1, 1)

        return 0

    lax.fori_loop(0, n_pairs, pair, 0)
    wait_store(0)
    wait_store(1)

    @pl.when(wid == NW - 1)
    def _():
        pltpu.sync_copy(tail_hbm, ot0.at[pl.ds(0, TAIL // 4)])
        pltpu.sync_copy(ot0.at[pl.ds(0, TAIL // 4)],
                        out_hbm.at[pl.ds(N_TILE_COLS * 32, TAIL // 4)])


_tr_sc = pl.kernel(
    _tr_body,
    out_type=jax.ShapeDtypeStruct((N_FEAT // 4, 128), jnp.float32),
    mesh=plsc.VectorSubcoreMesh(core_axis_name="c", subcore_axis_name="s"),
    compiler_params=pltpu.CompilerParams(needs_layout_passes=False,
                                         use_tc_tiling_on_sc=True),
    scratch_types=[
        pltpu.VMEM((N_FACTORS, 256), jnp.float32),
        pltpu.VMEM((N_FACTORS, 256), jnp.float32),
        pltpu.VMEM((64, 128), jnp.float32),
        pltpu.VMEM((64, 128), jnp.float32),
        pltpu.SemaphoreType.DMA,
        pltpu.SemaphoreType.DMA,
        pltpu.SemaphoreType.DMA,
        pltpu.SemaphoreType.DMA,
    ],
)


ROWS_PER_W = BATCH // NW
CHUNK = 64
N_CHUNKS = ROWS_PER_W // CHUNK
IDXC = CHUNK * N_FIELDS
GATHERS = IDXC // 128
X_PER_W = ROWS_PER_W * N_FIELDS


def _fm_body(x_hbm, emb_hbm, fc_hbm, w_hbm, b_hbm, out_hbm,
             xall, idx0, idx1, rows0, rows1, fcv0, fcv1,
             sqv, sqq, outv, wv_v, bv_v, sem0, sem1):
    cid = lax.axis_index("c")
    sid = lax.axis_index("s")
    wid = sid * NC + cid
    xbase = wid * X_PER_W

    pltpu.sync_copy(x_hbm.at[pl.ds(xbase, X_PER_W)], xall)
    pltpu.sync_copy(w_hbm, wv_v)
    pltpu.sync_copy(b_hbm, bv_v)
    Wv = wv_v[...]
    Bv = bv_v[...]

    iota16 = jnp.arange(16, dtype=jnp.int32)
    riota26 = iota16 * N_FIELDS
    riota17 = iota16 * 17

    idx_bufs = (idx0, idx1)
    rows_bufs = (rows0, rows1)
    fc_bufs = (fcv0, fcv1)
    sems = (sem0, sem1)

    def prep_fire(ci, p):
        idxP, rowsP, fcP, semP = idx_bufs[p], rows_bufs[p], fc_bufs[p], sems[p]

        def kbody(k, _):
            off = k * 16
            pos = off + iota16
            xv = xall[pl.ds(ci * IDXC + off, 16)]
            fld = lax.rem(pos, N_FIELDS)
            idxP[pl.ds(off, 16)] = xv + fld * FIELD_DIM
            return 0

        lax.fori_loop(0, IDXC // 16, kbody, 0)
        for j in range(GATHERS):
            isl = idxP.at[pl.ds(j * 128, 128)]
            pltpu.async_copy(emb_hbm.at[isl], rowsP.at[pl.ds(j * 128, 128)], semP)
            pltpu.async_copy(fc_hbm.at[isl], fcP.at[pl.ds(j * 128, 128)], semP)

    def drain(p):
        pltpu.make_async_copy(emb_hbm.at[pl.ds(0, IDXC)],
                              rows_bufs[p], sems[p]).wait()
        pltpu.make_async_copy(fc_hbm.at[pl.ds(0, IDXC)],
                              fc_bufs[p], sems[p]).wait()

    def compute(ci, p):
        rowsP, fcP = rows_bufs[p], fc_bufs[p]
        zf = jnp.zeros((16,), jnp.float32)

        def gbody(g, _):
            gb = g * 16

            def sbody(i, _):
                sb = (gb + i) * N_FIELDS
                a0, a1, a2, a3 = zf, zf, zf, zf
                q0, q1, q2, q3 = zf, zf, zf, zf
                for f in range(N_FIELDS):
                    e0 = rowsP[sb + f, pl.ds(0, 16)]
                    e1 = rowsP[sb + f, pl.ds(16, 16)]
                    if f % 2 == 0:
                        a0 = a0 + e0
                        a1 = a1 + e1
                        q0 = q0 + e0 * e0
                        q1 = q1 + e1 * e1
                    else:
                        a2 = a2 + e0
                        a3 = a3 + e1
                        q2 = q2 + e0 * e0
                        q3 = q3 + e1 * e1
                s0 = a0 + a2
                s1 = a1 + a3
                v = s0 * s0 + s1 * s1
                q = (q0 + q1) + (q2 + q3)
                sqv[pl.ds(i * 17, 16)] = v
                sqq[pl.ds(i * 17, 16)] = q
                return 0

            lax.fori_loop(0, 16, sbody, 0)

            nrm0, nrm1, qt0, qt1 = zf, zf, zf, zf
            for k in range(16):
                ik = riota17 + k
                if k % 2 == 0:
                    nrm0 = nrm0 + plsc.load_gather(sqv, [ik])
                    qt0 = qt0 + plsc.load_gather(sqq, [ik])
                else:
                    nrm1 = nrm1 + plsc.load_gather(sqv, [ik])
                    qt1 = qt1 + plsc.load_gather(sqq, [ik])

            rowb26 = riota26 + gb * N_FIELDS
            f0, f1 = zf, zf
            for f in range(N_FIELDS):
                e = plsc.load_gather(fcP, [rowb26 + f])
                if f % 2 == 0:
                    f0 = f0 + e
                else:
                    f1 = f1 + e

            res = ((f0 + f1) * Wv + Bv
                   + 0.5 * ((nrm0 + nrm1) - (qt0 + qt1)))
            outv[pl.ds(ci * CHUNK + g * 16, 16)] = res
            return 0

        lax.fori_loop(0, CHUNK // 16, gbody, 0)

    prep_fire(0, 0)
    for ci in range(N_CHUNKS):
        p = ci & 1
        if ci + 1 < N_CHUNKS:
            prep_fire(ci + 1, 1 - p)
        drain(p)
        compute(ci, p)

    pltpu.sync_copy(outv, out_hbm.at[pl.ds(wid * ROWS_PER_W, ROWS_PER_W)])


_fm_sc = pl.kernel(
    _fm_body,
    out_type=jax.ShapeDtypeStruct((BATCH,), jnp.float32),
    mesh=plsc.VectorSubcoreMesh(core_axis_name="c", subcore_axis_name="s"),
    compiler_params=pltpu.CompilerParams(needs_layout_passes=False,
                                         use_tc_tiling_on_sc=False),
    scratch_types=[
        pltpu.VMEM((X_PER_W,), jnp.int32),
        pltpu.VMEM((IDXC,), jnp.int32),
        pltpu.VMEM((IDXC,), jnp.int32),
        pltpu.VMEM((IDXC, N_FACTORS), jnp.float32),
        pltpu.VMEM((IDXC, N_FACTORS), jnp.float32),
        pltpu.VMEM((IDXC,), jnp.float32),
        pltpu.VMEM((IDXC,), jnp.float32),
        pltpu.VMEM((16 * 17,), jnp.float32),
        pltpu.VMEM((16 * 17,), jnp.float32),
        pltpu.VMEM((ROWS_PER_W,), jnp.float32),
        pltpu.VMEM((16,), jnp.float32),
        pltpu.VMEM((16,), jnp.float32),
        pltpu.SemaphoreType.DMA,
        pltpu.SemaphoreType.DMA,
    ],
)


@jax.jit
def kernel(x, embedding, fc, W, b):
    x_flat = x.astype(jnp.int32).reshape(-1)
    tail4 = embedding[N_TILE_COLS * 128:].reshape(TAIL // 4, 128)
    emb4 = _tr_sc(embedding.T, tail4)
    emb_rows = emb4.reshape(N_FEAT, N_FACTORS)
    fc_flat = fc.reshape(-1).astype(jnp.float32)
    wv = jnp.full((16,), W[0, 0], dtype=jnp.float32)
    bv = jnp.full((16,), b[0], dtype=jnp.float32)
    return _fm_sc(x_flat, emb_rows, fc_flat, wv, bv)
